# Initial kernel scaffold; baseline (speedup 1.0000x reference)
import jax
import jax.numpy as jnp
from jax.experimental import pallas as pl


def kernel(x, edge_index, edge_attr, batch, node_W, node_b, edge_W, edge_b, src_W, dst_W, lin_edge_W, mlp1_W, mlp2_W, mlp3_W, mlp4_W, bn_gamma, bn_beta, ain_W, ain_b, aout_W, aout_b):
    raise NotImplementedError("write your pallas kernel here")


# speedup vs baseline: 3.2716x; 3.2716x over previous
import functools

import jax
import jax.numpy as jnp
from jax import lax
from jax.experimental import pallas as pl
from jax.experimental.pallas import tpu as pltpu
from jax.experimental.pallas import tpu_sc as plsc

N = 50000
E = 800000
G = 64
D_IN = 128
D_EDGE = 16
HID = 32
OUT = 64
MLP_HID = 128
ACT = 16
EPS = 1e-7

NQ = 4
NHALF = N // 2
NPAD = 25088
ROWS_PER_TILE = NPAD // 16
TBL_SPLIT = 3128
TBL_LAST = N - 15 * TBL_SPLIT
CH = 128
NCHUNKS = E // CH
NJ = 392


_K1_TILE = 2000


def _k1_body(x_ref, nw_ref, nb_ref, sw_ref, dw_ref, hsq_ref, hd_ref, mx_ref,
             acc_ref):
    t = pl.program_id(0)
    h = jnp.dot(x_ref[...], nw_ref[...], preferred_element_type=jnp.float32)
    h = h + nb_ref[...]
    hs = jnp.dot(h, sw_ref[...], preferred_element_type=jnp.float32)
    hd = jnp.dot(h, dw_ref[...], preferred_element_type=jnp.float32)
    hd_ref[...] = hd
    for q in range(NQ):
        hsq_ref[q] = hs[:, 16 * q:16 * (q + 1)]
    m = jnp.max(hs)

    @pl.when(t == 0)
    def _():
        acc_ref[0] = m

    @pl.when(t > 0)
    def _():
        acc_ref[0] = jnp.maximum(acc_ref[0], m)

    @pl.when(t == pl.num_programs(0) - 1)
    def _():
        mx_ref[0, 0] = acc_ref[0]


def _node_encode(x, node_W, node_b, src_W, dst_W):
    grid = (N // _K1_TILE,)
    hsq, hd, mx = pl.pallas_call(
        _k1_body,
        grid=grid,
        in_specs=[
            pl.BlockSpec((_K1_TILE, D_IN), lambda t: (t, 0)),
            pl.BlockSpec((D_IN, HID), lambda t: (0, 0)),
            pl.BlockSpec((HID,), lambda t: (0,)),
            pl.BlockSpec((HID, OUT), lambda t: (0, 0)),
            pl.BlockSpec((HID, OUT), lambda t: (0, 0)),
        ],
        out_specs=[
            pl.BlockSpec((NQ, _K1_TILE, 16), lambda t: (0, t, 0)),
            pl.BlockSpec((_K1_TILE, OUT), lambda t: (t, 0)),
            pl.BlockSpec(memory_space=pltpu.SMEM),
        ],
        out_shape=[
            jax.ShapeDtypeStruct((NQ, N, 16), jnp.float32),
            jax.ShapeDtypeStruct((N, OUT), jnp.float32),
            jax.ShapeDtypeStruct((1, 1), jnp.float32),
        ],
        scratch_shapes=[pltpu.SMEM((1,), jnp.float32)],
    )(x, node_W, node_b, src_W, dst_W)
    return hsq, hd, mx


_K4_TILE = 2000


def _k4_body(ea_ref, w_ref, b_ref, epq_ref, mx_ref, acc_ref):
    t = pl.program_id(0)
    ep = jnp.dot(ea_ref[...], w_ref[...], preferred_element_type=jnp.float32)
    ep = ep + b_ref[...]
    for q in range(NQ):
        epq_ref[q] = ep[:, 16 * q:16 * (q + 1)]
    m = jnp.max(ep)

    @pl.when(t == 0)
    def _():
        acc_ref[0] = m

    @pl.when(t > 0)
    def _():
        acc_ref[0] = jnp.maximum(acc_ref[0], m)

    @pl.when(t == pl.num_programs(0) - 1)
    def _():
        mx_ref[0, 0] = acc_ref[0]


def _edge_encode(edge_attr, w2, b2):
    grid = (E // _K4_TILE,)
    epq, mx = pl.pallas_call(
        _k4_body,
        grid=grid,
        in_specs=[
            pl.BlockSpec((_K4_TILE, D_EDGE), lambda t: (t, 0)),
            pl.BlockSpec((D_EDGE, OUT), lambda t: (0, 0)),
            pl.BlockSpec((OUT,), lambda t: (0,)),
        ],
        out_specs=[
            pl.BlockSpec((NQ, _K4_TILE, 16), lambda t: (0, t, 0)),
            pl.BlockSpec(memory_space=pltpu.SMEM),
        ],
        out_shape=[
            jax.ShapeDtypeStruct((NQ, E, 16), jnp.float32),
            jax.ShapeDtypeStruct((1, 1), jnp.float32),
        ],
        scratch_shapes=[pltpu.SMEM((1,), jnp.float32)],
    )(edge_attr, w2, b2)
    return epq, mx


def _k5_body(hsq, epq, src_h, dst_h, mvec_h, zeros_h, out_h,
             table, accum, mbuf,
             sidx0, sidx1, didx0, didx1, ebuf0, ebuf1,
             gbuf0, gbuf1, vbuf0, vbuf1, cidx0, cidx1,
             sem_i0, sem_i1, sem_g, sem_s0, sem_s1):
    cid = lax.axis_index("c")
    sid = lax.axis_index("s")
    base = cid * NHALF

    sidx = (sidx0, sidx1)
    didx = (didx0, didx1)
    ebuf = (ebuf0, ebuf1)
    gbuf = (gbuf0, gbuf1)
    vbuf = (vbuf0, vbuf1)
    cidx = (cidx0, cidx1)
    sem_i = (sem_i0, sem_i1)
    sem_s = (sem_s0, sem_s1)

    pltpu.sync_copy(mvec_h, mbuf)
    mv = mbuf[...]
    lane = lax.iota(jnp.int32, 16)

    def chunk_off(j):
        cidr = sid + j * 16
        return jnp.minimum(cidr, NCHUNKS - 1) * CH, cidr < NCHUNKS

    for q in range(NQ):
        pltpu.sync_copy(zeros_h.at[pl.ds(sid * ROWS_PER_TILE, ROWS_PER_TILE)],
                        accum.at[pl.ds(sid * ROWS_PER_TILE, ROWS_PER_TILE)])
        @pl.when(sid < 15)
        def _():
            pltpu.sync_copy(hsq.at[q, pl.ds(sid * TBL_SPLIT, TBL_SPLIT)],
                            table.at[pl.ds(sid * TBL_SPLIT, TBL_SPLIT)])

        @pl.when(sid == 15)
        def _():
            pltpu.sync_copy(hsq.at[q, pl.ds(15 * TBL_SPLIT, TBL_LAST)],
                            table.at[pl.ds(15 * TBL_SPLIT, TBL_LAST)])
        plsc.subcore_barrier()

        def issue_in(j, b):
            off, _ = chunk_off(j)
            pltpu.async_copy(src_h.at[pl.ds(off, CH)], sidx[b], sem_i[b])
            pltpu.async_copy(dst_h.at[pl.ds(off, CH)], didx[b], sem_i[b])
            pltpu.async_copy(epq.at[q, pl.ds(off, CH)], ebuf[b], sem_i[b])

        def wait_in(j, b):
            off, _ = chunk_off(j)
            pltpu.make_async_copy(src_h.at[pl.ds(off, CH)], sidx[b],
                                  sem_i[b]).wait()
            pltpu.make_async_copy(dst_h.at[pl.ds(off, CH)], didx[b],
                                  sem_i[b]).wait()
            pltpu.make_async_copy(epq.at[q, pl.ds(off, CH)], ebuf[b],
                                  sem_i[b]).wait()

        issue_in(0, 0)
        issue_in(1, 1)
        wait_in(0, 0)
        pltpu.async_copy(table.at[sidx[0]], gbuf[0], sem_g)

        def step(j, b):
            nb = 1 - b
            off, valid = chunk_off(j)
            @pl.when(j >= 2)
            def _():
                pltpu.make_async_copy(vbuf[b], accum.at[cidx[b]],
                                      sem_s[b]).wait()

            pltpu.make_async_copy(table.at[sidx[b]], gbuf[b], sem_g).wait()

            shift = jnp.where(valid, 0, N + 16)
            vs = [didx[b][pl.ds(i * 16, 16)] + shift for i in range(CH // 16)]
            lidxs = [jnp.where((v >= base) & (v < base + NHALF),
                               v - base, NHALF + lane) for v in vs]
            for i in range(CH // 16):
                cidx[b][pl.ds(i * 16, 16)] = lidxs[i]

            def vstep(i, _):
                es = [i * 8 + u for u in range(8)]
                gs = [gbuf[b][e] for e in es]
                eas = [ebuf[b][e] for e in es]
                msgs = [jnp.maximum(g + a, 0.0) + EPS
                        for g, a in zip(gs, eas)]
                exs = [jnp.exp(m - mv) for m in msgs]
                ems = [x * m for x, m in zip(exs, msgs)]
                for u, e in enumerate(es):
                    vbuf[b][e, 0:16] = exs[u]
                for u, e in enumerate(es):
                    vbuf[b][e, 16:32] = ems[u]
                return 0

            lax.fori_loop(0, CH // 8, vstep, 0)

            pltpu.async_copy(vbuf[b], accum.at[cidx[b]], sem_s[b], add=True)

            @pl.when(j + 1 < NJ)
            def _():
                wait_in(j + 1, nb)
                pltpu.async_copy(table.at[sidx[nb]], gbuf[nb], sem_g)

            @pl.when(j + 2 < NJ)
            def _():
                issue_in(j + 2, b)

        def loop_body(g2, _):
            step(2 * g2, 0)
            step(2 * g2 + 1, 1)
            return 0

        lax.fori_loop(0, NJ // 2, loop_body, 0)

        pltpu.make_async_copy(vbuf[0], accum.at[cidx[0]], sem_s[0]).wait()
        pltpu.make_async_copy(vbuf[1], accum.at[cidx[1]], sem_s[1]).wait()
        plsc.subcore_barrier()
        pltpu.sync_copy(accum.at[pl.ds(sid * ROWS_PER_TILE, ROWS_PER_TILE)],
                        out_h.at[q, cid, pl.ds(sid * ROWS_PER_TILE,
                                               ROWS_PER_TILE)])
        plsc.subcore_barrier()


def _softmax_aggregate(hsq, epq, src, dst, mvec, zeros):
    mesh = plsc.VectorSubcoreMesh(core_axis_name="c", subcore_axis_name="s")
    kfun = pl.kernel(
        _k5_body,
        out_type=jax.ShapeDtypeStruct((NQ, 2, NPAD, 32), jnp.float32),
        mesh=mesh,
        compiler_params=pltpu.CompilerParams(use_tc_tiling_on_sc=False),
        scratch_types=[
            pltpu.VMEM_SHARED((N, 16), jnp.float32),
            pltpu.VMEM_SHARED((NPAD, 32), jnp.float32),
            pltpu.VMEM((16,), jnp.float32),
            pltpu.VMEM((CH,), jnp.int32),
            pltpu.VMEM((CH,), jnp.int32),
            pltpu.VMEM((CH,), jnp.int32),
            pltpu.VMEM((CH,), jnp.int32),
            pltpu.VMEM((CH, 16), jnp.float32),
            pltpu.VMEM((CH, 16), jnp.float32),
            pltpu.VMEM((CH, 16), jnp.float32),
            pltpu.VMEM((CH, 16), jnp.float32),
            pltpu.VMEM((CH, 32), jnp.float32),
            pltpu.VMEM((CH, 32), jnp.float32),
            pltpu.VMEM((CH,), jnp.int32),
            pltpu.VMEM((CH,), jnp.int32),
            pltpu.SemaphoreType.DMA,
            pltpu.SemaphoreType.DMA,
            pltpu.SemaphoreType.DMA,
            pltpu.SemaphoreType.DMA,
            pltpu.SemaphoreType.DMA,
        ],
    )
    return kfun(hsq, epq, src, dst, mvec, zeros)


_K6A_TILE = 1000


def _k6a_body(o_ref, hd_ref, out_ref):
    hd = hd_ref[...]
    res = jnp.zeros_like(hd)
    cols = []
    for q in range(NQ):
        den = o_ref[q, 0, :, 0:16]
        num = o_ref[q, 0, :, 16:32]
        cols.append(jnp.where(den > 0.0, num / den, 0.0))
    res = jnp.concatenate(cols, axis=1) + hd
    out_ref[...] = res


def _combine(o, hd):
    nt = NHALF // _K6A_TILE
    return pl.pallas_call(
        _k6a_body,
        grid=(2, nt),
        in_specs=[
            pl.BlockSpec((NQ, 1, _K6A_TILE, 32), lambda c, t: (0, c, t, 0)),
            pl.BlockSpec((_K6A_TILE, OUT), lambda c, t: (c * nt + t, 0)),
        ],
        out_specs=pl.BlockSpec((_K6A_TILE, OUT), lambda c, t: (c * nt + t, 0)),
        out_shape=jax.ShapeDtypeStruct((N, OUT), jnp.float32),
    )(o, hd)


_K6B_TILE = 2000
_K6B_NT = N // _K6B_TILE


def _k6b_body(out_ref, batch_ref, w1_ref, w2_ref, w3_ref, w4_ref,
              gam_ref, bet_ref, ainw_ref, ainb_ref, aoutw_ref, aoutb_ref,
              a_ref, hh_ref, s_ref, ss_ref, pool_ref, cnt_ref):
    ph = pl.program_id(0)
    t = pl.program_id(1)
    nt = pl.num_programs(1)
    row = pl.ds(t * _K6B_TILE, _K6B_TILE)
    inv_n = 1.0 / N

    def bn(x, l):
        mu = s_ref[l] * inv_n
        var = ss_ref[l] * inv_n - mu * mu
        xn = (x - mu) * lax.rsqrt(var + 1e-5)
        return jnp.maximum(xn * gam_ref[l] + bet_ref[l], 0.0)

    def stats(l, y):
        @pl.when(t == 0)
        def _():
            s_ref[l] = jnp.sum(y, axis=0)
            ss_ref[l] = jnp.sum(y * y, axis=0)

        @pl.when(t > 0)
        def _():
            s_ref[l] = s_ref[l] + jnp.sum(y, axis=0)
            ss_ref[l] = ss_ref[l] + jnp.sum(y * y, axis=0)

    @pl.when(ph == 0)
    def _():
        y = jnp.dot(out_ref[...], w1_ref[...],
                    preferred_element_type=jnp.float32)
        hh_ref[row, :] = y
        stats(0, y)

    @pl.when(ph == 1)
    def _():
        y = jnp.dot(bn(hh_ref[row, :], 0), w2_ref[...],
                    preferred_element_type=jnp.float32)
        hh_ref[row, :] = y
        stats(1, y)

    @pl.when(ph == 2)
    def _():
        y = jnp.dot(bn(hh_ref[row, :], 1), w3_ref[...],
                    preferred_element_type=jnp.float32)
        hh_ref[row, :] = y
        stats(2, y)

    @pl.when(ph == 3)
    def _():
        p = jnp.dot(bn(hh_ref[row, :], 2), w4_ref[...],
                    preferred_element_type=jnp.float32)
        gids = lax.broadcasted_iota(jnp.int32, (_K6B_TILE, G), 1)
        onehot = (batch_ref[...] == gids).astype(jnp.float32)
        psum = jnp.dot(onehot.T, p, preferred_element_type=jnp.float32)
        csum = jnp.sum(onehot, axis=0)

        @pl.when(t == 0)
        def _():
            pool_ref[...] = psum
            cnt_ref[0] = csum

        @pl.when(t > 0)
        def _():
            pool_ref[...] = pool_ref[...] + psum
            cnt_ref[0] = cnt_ref[0] + csum

        @pl.when(t == nt - 1)
        def _():
            cnt = jnp.maximum(cnt_ref[0], 1.0)
            mol = pool_ref[...] / cnt[:, None]
            z = jnp.maximum(
                jnp.dot(mol, ainw_ref[...],
                        preferred_element_type=jnp.float32) + ainb_ref[...],
                0.0)
            a_ref[...] = jnp.tanh(
                jnp.dot(z, aoutw_ref[...],
                        preferred_element_type=jnp.float32) + aoutb_ref[...])


def _mlp_head(out, batch2d, mlp1_W, mlp2_W, mlp3_W, mlp4_W, bn_gamma, bn_beta,
              ain_W, ain_b, aout_W, aout_b):
    return pl.pallas_call(
        _k6b_body,
        grid=(4, _K6B_NT),
        in_specs=[
            pl.BlockSpec((_K6B_TILE, OUT), lambda ph, t: (t, 0)),
            pl.BlockSpec((_K6B_TILE, 1), lambda ph, t: (t, 0)),
            pl.BlockSpec((OUT, MLP_HID), lambda ph, t: (0, 0)),
            pl.BlockSpec((MLP_HID, MLP_HID), lambda ph, t: (0, 0)),
            pl.BlockSpec((MLP_HID, MLP_HID), lambda ph, t: (0, 0)),
            pl.BlockSpec((MLP_HID, OUT), lambda ph, t: (0, 0)),
            pl.BlockSpec((3, MLP_HID), lambda ph, t: (0, 0)),
            pl.BlockSpec((3, MLP_HID), lambda ph, t: (0, 0)),
            pl.BlockSpec((OUT, 16), lambda ph, t: (0, 0)),
            pl.BlockSpec((16,), lambda ph, t: (0,)),
            pl.BlockSpec((16, ACT), lambda ph, t: (0, 0)),
            pl.BlockSpec((ACT,), lambda ph, t: (0,)),
        ],
        out_specs=pl.BlockSpec((G, ACT), lambda ph, t: (0, 0)),
        out_shape=jax.ShapeDtypeStruct((G, ACT), jnp.float32),
        scratch_shapes=[
            pltpu.VMEM((N, MLP_HID), jnp.float32),
            pltpu.VMEM((3, MLP_HID), jnp.float32),
            pltpu.VMEM((3, MLP_HID), jnp.float32),
            pltpu.VMEM((G, OUT), jnp.float32),
            pltpu.VMEM((1, G), jnp.float32),
        ],
    )(out, batch2d, mlp1_W, mlp2_W, mlp3_W, mlp4_W, bn_gamma, bn_beta,
      ain_W, ain_b, aout_W, aout_b)


def kernel(x, edge_index, edge_attr, batch, node_W, node_b, edge_W, edge_b,
           src_W, dst_W, lin_edge_W, mlp1_W, mlp2_W, mlp3_W, mlp4_W, bn_gamma,
           bn_beta, ain_W, ain_b, aout_W, aout_b):
    hsq, hd, maxh = _node_encode(x, node_W, node_b, src_W, dst_W)

    w2 = edge_W @ lin_edge_W
    b2 = edge_b @ lin_edge_W
    epq, maxe = _edge_encode(edge_attr, w2, b2)

    m_shift = jnp.maximum(maxh[0, 0] + maxe[0, 0], 0.0) + EPS
    mvec = jnp.full((16,), m_shift, jnp.float32)
    zeros = jnp.zeros((NPAD, 32), jnp.float32)

    src = edge_index[0]
    dst = edge_index[1]
    o = _softmax_aggregate(hsq, epq, src, dst, mvec, zeros)

    out = _combine(o, hd)
    batch2d = batch.reshape(N, 1).astype(jnp.int32)
    return _mlp_head(out, batch2d, mlp1_W, mlp2_W, mlp3_W, mlp4_W, bn_gamma,
                     bn_beta, ain_W, ain_b, aout_W, aout_b)


# speedup vs baseline: 6.2238x; 1.9023x over previous
import functools

import jax
import jax.numpy as jnp
from jax import lax
from jax.experimental import pallas as pl
from jax.experimental.pallas import tpu as pltpu
from jax.experimental.pallas import tpu_sc as plsc

N = 50000
E = 800000
G = 64
D_IN = 128
D_EDGE = 16
HID = 32
OUT = 64
MLP_HID = 128
ACT = 16
EPS = 1e-7

NQ = 4
NPAD = 50176
ROWS_PER_TILE = NPAD // 16
CH = 128
NCHUNKS = E // CH
NJ = 392
EP8 = E // 8


_K1_TILE = 2000


def _k1_body(x_ref, nw_ref, nb_ref, sw_ref, dw_ref, hsq_ref, hd_ref, mx_ref,
             acc_ref):
    t = pl.program_id(0)
    h = jnp.dot(x_ref[...], nw_ref[...], preferred_element_type=jnp.float32)
    h = h + nb_ref[...]
    hs = jnp.dot(h, sw_ref[...], preferred_element_type=jnp.float32)
    hd = jnp.dot(h, dw_ref[...], preferred_element_type=jnp.float32)
    hd_ref[...] = hd
    for q in range(NQ):
        hsq_ref[q] = hs[:, 16 * q:16 * (q + 1)]
    m = jnp.max(hs)

    @pl.when(t == 0)
    def _():
        acc_ref[0] = m

    @pl.when(t > 0)
    def _():
        acc_ref[0] = jnp.maximum(acc_ref[0], m)

    @pl.when(t == pl.num_programs(0) - 1)
    def _():
        mx_ref[0, 0] = acc_ref[0]


def _node_encode(x, node_W, node_b, src_W, dst_W):
    grid = (N // _K1_TILE,)
    hsq, hd, mx = pl.pallas_call(
        _k1_body,
        grid=grid,
        in_specs=[
            pl.BlockSpec((_K1_TILE, D_IN), lambda t: (t, 0)),
            pl.BlockSpec((D_IN, HID), lambda t: (0, 0)),
            pl.BlockSpec((HID,), lambda t: (0,)),
            pl.BlockSpec((HID, OUT), lambda t: (0, 0)),
            pl.BlockSpec((HID, OUT), lambda t: (0, 0)),
        ],
        out_specs=[
            pl.BlockSpec((NQ, _K1_TILE, 16), lambda t: (0, t, 0)),
            pl.BlockSpec((_K1_TILE, OUT), lambda t: (t, 0)),
            pl.BlockSpec(memory_space=pltpu.SMEM),
        ],
        out_shape=[
            jax.ShapeDtypeStruct((NQ, N, 16), jnp.float32),
            jax.ShapeDtypeStruct((N, OUT), jnp.float32),
            jax.ShapeDtypeStruct((1, 1), jnp.float32),
        ],
        scratch_shapes=[pltpu.SMEM((1,), jnp.float32)],
    )(x, node_W, node_b, src_W, dst_W)
    return hsq, hd, mx


_K4_TILE8 = 1000


def _k4_body(ea_ref, w_ref, b_ref, epq_ref, mx_ref, acc_ref):
    t = pl.program_id(0)
    x = ea_ref[...]
    m = None
    for q in range(NQ):
        ep = jnp.dot(x, w_ref[q], preferred_element_type=jnp.float32)
        ep = ep + b_ref[q]
        epq_ref[q] = ep
        mq = jnp.max(ep)
        m = mq if m is None else jnp.maximum(m, mq)

    @pl.when(t == 0)
    def _():
        acc_ref[0] = m

    @pl.when(t > 0)
    def _():
        acc_ref[0] = jnp.maximum(acc_ref[0], m)

    @pl.when(t == pl.num_programs(0) - 1)
    def _():
        mx_ref[0, 0] = acc_ref[0]


def _edge_encode(edge_attr, w2, b2):
    wblk = jnp.zeros((NQ, 128, 128), jnp.float32)
    for q in range(NQ):
        for u in range(8):
            wblk = wblk.at[q, 16 * u:16 * (u + 1),
                           16 * u:16 * (u + 1)].set(w2[:, 16 * q:16 * (q + 1)])
    bpack = jnp.tile(b2.reshape(NQ, 16), (1, 8))

    ea8 = edge_attr.reshape(EP8, 128)
    grid = (EP8 // _K4_TILE8,)
    epq, mx = pl.pallas_call(
        _k4_body,
        grid=grid,
        in_specs=[
            pl.BlockSpec((_K4_TILE8, 128), lambda t: (t, 0)),
            pl.BlockSpec((NQ, 128, 128), lambda t: (0, 0, 0)),
            pl.BlockSpec((NQ, 128), lambda t: (0, 0)),
        ],
        out_specs=[
            pl.BlockSpec((NQ, _K4_TILE8, 128), lambda t: (0, t, 0)),
            pl.BlockSpec(memory_space=pltpu.SMEM),
        ],
        out_shape=[
            jax.ShapeDtypeStruct((NQ, EP8, 128), jnp.float32),
            jax.ShapeDtypeStruct((1, 1), jnp.float32),
        ],
        scratch_shapes=[pltpu.SMEM((1,), jnp.float32)],
    )(ea8, wblk, bpack)
    return epq, mx


def _k5_body(hsf, epq, src_h, dst_h, mvec_h, zeros_h, out_h,
             accum, mbuf,
             sidx0, sidx1, didx0, didx1, ebuf0, ebuf1,
             gbuf0, gbuf1, vbuf0, vbuf1, cidx0, cidx1,
             sem_i0, sem_i1, sem_g, sem_s0, sem_s1):
    cid = lax.axis_index("c")
    sid = lax.axis_index("s")

    sidx = (sidx0, sidx1)
    didx = (didx0, didx1)
    ebuf = (ebuf0, ebuf1)
    gbuf = (gbuf0, gbuf1)
    vbuf = (vbuf0, vbuf1)
    cidx = (cidx0, cidx1)
    sem_i = (sem_i0, sem_i1)
    sem_s = (sem_s0, sem_s1)

    pltpu.sync_copy(mvec_h, mbuf)
    mv = mbuf[...]
    lane = lax.iota(jnp.int32, 16)

    def chunk_off(j):
        cidr = sid + j * 16
        return jnp.minimum(cidr, NCHUNKS - 1) * CH, cidr < NCHUNKS

    for r in range(2):
        q = 2 * r + cid
        qoff = q * N
        pltpu.sync_copy(zeros_h.at[pl.ds(sid * ROWS_PER_TILE, ROWS_PER_TILE)],
                        accum.at[pl.ds(sid * ROWS_PER_TILE, ROWS_PER_TILE)])
        plsc.subcore_barrier()

        def issue_in(j, b):
            off, _ = chunk_off(j)
            pltpu.async_copy(src_h.at[pl.ds(off, CH)], sidx[b], sem_i[b])
            pltpu.async_copy(dst_h.at[pl.ds(off, CH)], didx[b], sem_i[b])
            pltpu.async_copy(epq.at[q, pl.ds(off // 8, CH // 8)], ebuf[b],
                             sem_i[b])

        def wait_in(j, b):
            off, _ = chunk_off(j)
            pltpu.make_async_copy(src_h.at[pl.ds(off, CH)], sidx[b],
                                  sem_i[b]).wait()
            pltpu.make_async_copy(dst_h.at[pl.ds(off, CH)], didx[b],
                                  sem_i[b]).wait()
            pltpu.make_async_copy(epq.at[q, pl.ds(off // 8, CH // 8)],
                                  ebuf[b], sem_i[b]).wait()

        def arm_gather(b):
            for i in range(CH // 16):
                sidx[b][pl.ds(i * 16, 16)] = \
                    sidx[b][pl.ds(i * 16, 16)] + qoff
            pltpu.async_copy(hsf.at[sidx[b]], gbuf[b], sem_g)

        issue_in(0, 0)
        issue_in(1, 1)
        wait_in(0, 0)
        arm_gather(0)

        def step(j, b):
            nb = 1 - b
            off, valid = chunk_off(j)
            @pl.when(j >= 2)
            def _():
                pltpu.make_async_copy(vbuf[b], accum.at[cidx[b]],
                                      sem_s[b]).wait()

            pltpu.make_async_copy(hsf.at[sidx[b]], gbuf[b], sem_g).wait()

            shift = jnp.where(valid, 0, N + 16)
            vs = [didx[b][pl.ds(i * 16, 16)] + shift
                  for i in range(CH // 16)]
            lidxs = [jnp.where(v < N, v, N + lane) for v in vs]
            for i in range(CH // 16):
                cidx[b][pl.ds(i * 16, 16)] = lidxs[i]

            def vstep(i, _):
                gs = [gbuf[b][i * 8 + u] for u in range(8)]
                eas = [ebuf[b][i, pl.ds(u * 16, 16)] for u in range(8)]
                msgs = [jnp.maximum(g + a, 0.0) + EPS
                        for g, a in zip(gs, eas)]
                exs = [jnp.exp(m - mv) for m in msgs]
                ems = [x * m for x, m in zip(exs, msgs)]
                for u in range(8):
                    vbuf[b][i * 8 + u, 0:16] = exs[u]
                for u in range(8):
                    vbuf[b][i * 8 + u, 16:32] = ems[u]
                return 0

            lax.fori_loop(0, CH // 8, vstep, 0)

            pltpu.async_copy(vbuf[b], accum.at[cidx[b]], sem_s[b], add=True)

            @pl.when(j + 1 < NJ)
            def _():
                wait_in(j + 1, nb)
                arm_gather(nb)

            @pl.when(j + 2 < NJ)
            def _():
                issue_in(j + 2, b)

        def loop_body(g2, _):
            step(2 * g2, 0)
            step(2 * g2 + 1, 1)
            return 0

        lax.fori_loop(0, NJ // 2, loop_body, 0)

        pltpu.make_async_copy(vbuf[0], accum.at[cidx[0]], sem_s[0]).wait()
        pltpu.make_async_copy(vbuf[1], accum.at[cidx[1]], sem_s[1]).wait()
        plsc.subcore_barrier()
        pltpu.sync_copy(accum.at[pl.ds(sid * ROWS_PER_TILE, ROWS_PER_TILE)],
                        out_h.at[q, pl.ds(sid * ROWS_PER_TILE,
                                          ROWS_PER_TILE)])
        plsc.subcore_barrier()


def _softmax_aggregate(hsq, epq, src, dst, mvec, zeros):
    mesh = plsc.VectorSubcoreMesh(core_axis_name="c", subcore_axis_name="s")
    kfun = pl.kernel(
        _k5_body,
        out_type=jax.ShapeDtypeStruct((NQ, NPAD, 32), jnp.float32),
        mesh=mesh,
        compiler_params=pltpu.CompilerParams(use_tc_tiling_on_sc=False),
        scratch_types=[
            pltpu.VMEM_SHARED((NPAD, 32), jnp.float32),
            pltpu.VMEM((16,), jnp.float32),
            pltpu.VMEM((CH,), jnp.int32),
            pltpu.VMEM((CH,), jnp.int32),
            pltpu.VMEM((CH,), jnp.int32),
            pltpu.VMEM((CH,), jnp.int32),
            pltpu.VMEM((CH // 8, 128), jnp.float32),
            pltpu.VMEM((CH // 8, 128), jnp.float32),
            pltpu.VMEM((CH, 16), jnp.float32),
            pltpu.VMEM((CH, 16), jnp.float32),
            pltpu.VMEM((CH, 32), jnp.float32),
            pltpu.VMEM((CH, 32), jnp.float32),
            pltpu.VMEM((CH,), jnp.int32),
            pltpu.VMEM((CH,), jnp.int32),
            pltpu.SemaphoreType.DMA,
            pltpu.SemaphoreType.DMA,
            pltpu.SemaphoreType.DMA,
            pltpu.SemaphoreType.DMA,
            pltpu.SemaphoreType.DMA,
        ],
    )
    hsf = hsq.reshape(NQ * N, 16)
    return kfun(hsf, epq, src, dst, mvec, zeros)


_K6A_TILE = 2000


def _k6a_body(o_ref, hd_ref, out_ref):
    cols = []
    for q in range(NQ):
        den = o_ref[q, :, 0:16]
        num = o_ref[q, :, 16:32]
        cols.append(jnp.where(den > 0.0, num / den, 0.0))
    res = jnp.concatenate(cols, axis=1) + hd_ref[...]
    out_ref[...] = res


def _combine(o, hd):
    nt = N // _K6A_TILE
    return pl.pallas_call(
        _k6a_body,
        grid=(nt,),
        in_specs=[
            pl.BlockSpec((NQ, _K6A_TILE, 32), lambda t: (0, t, 0)),
            pl.BlockSpec((_K6A_TILE, OUT), lambda t: (t, 0)),
        ],
        out_specs=pl.BlockSpec((_K6A_TILE, OUT), lambda t: (t, 0)),
        out_shape=jax.ShapeDtypeStruct((N, OUT), jnp.float32),
    )(o, hd)


_K6B_TILE = 2000
_K6B_NT = N // _K6B_TILE


def _k6b_body(out_ref, batch_ref, w1_ref, w2_ref, w3_ref, w4_ref,
              gam_ref, bet_ref, ainw_ref, ainb_ref, aoutw_ref, aoutb_ref,
              a_ref, hh_ref, s_ref, ss_ref, pool_ref, cnt_ref):
    ph = pl.program_id(0)
    t = pl.program_id(1)
    nt = pl.num_programs(1)
    row = pl.ds(t * _K6B_TILE, _K6B_TILE)
    inv_n = 1.0 / N

    def bn(x, l):
        mu = s_ref[l] * inv_n
        var = ss_ref[l] * inv_n - mu * mu
        xn = (x - mu) * lax.rsqrt(var + 1e-5)
        return jnp.maximum(xn * gam_ref[l] + bet_ref[l], 0.0)

    def stats(l, y):
        @pl.when(t == 0)
        def _():
            s_ref[l] = jnp.sum(y, axis=0)
            ss_ref[l] = jnp.sum(y * y, axis=0)

        @pl.when(t > 0)
        def _():
            s_ref[l] = s_ref[l] + jnp.sum(y, axis=0)
            ss_ref[l] = ss_ref[l] + jnp.sum(y * y, axis=0)

    @pl.when(ph == 0)
    def _():
        y = jnp.dot(out_ref[...], w1_ref[...],
                    preferred_element_type=jnp.float32)
        hh_ref[row, :] = y
        stats(0, y)

    @pl.when(ph == 1)
    def _():
        y = jnp.dot(bn(hh_ref[row, :], 0), w2_ref[...],
                    preferred_element_type=jnp.float32)
        hh_ref[row, :] = y
        stats(1, y)

    @pl.when(ph == 2)
    def _():
        y = jnp.dot(bn(hh_ref[row, :], 1), w3_ref[...],
                    preferred_element_type=jnp.float32)
        hh_ref[row, :] = y
        stats(2, y)

    @pl.when(ph == 3)
    def _():
        p = jnp.dot(bn(hh_ref[row, :], 2), w4_ref[...],
                    preferred_element_type=jnp.float32)
        gids = lax.broadcasted_iota(jnp.int32, (_K6B_TILE, G), 1)
        onehot = (batch_ref[...] == gids).astype(jnp.float32)
        psum = jnp.dot(onehot.T, p, preferred_element_type=jnp.float32)
        csum = jnp.sum(onehot, axis=0)

        @pl.when(t == 0)
        def _():
            pool_ref[...] = psum
            cnt_ref[0] = csum

        @pl.when(t > 0)
        def _():
            pool_ref[...] = pool_ref[...] + psum
            cnt_ref[0] = cnt_ref[0] + csum

        @pl.when(t == nt - 1)
        def _():
            cnt = jnp.maximum(cnt_ref[0], 1.0)
            mol = pool_ref[...] / cnt[:, None]
            z = jnp.maximum(
                jnp.dot(mol, ainw_ref[...],
                        preferred_element_type=jnp.float32) + ainb_ref[...],
                0.0)
            a_ref[...] = jnp.tanh(
                jnp.dot(z, aoutw_ref[...],
                        preferred_element_type=jnp.float32) + aoutb_ref[...])


def _mlp_head(out, batch2d, mlp1_W, mlp2_W, mlp3_W, mlp4_W, bn_gamma, bn_beta,
              ain_W, ain_b, aout_W, aout_b):
    return pl.pallas_call(
        _k6b_body,
        grid=(4, _K6B_NT),
        in_specs=[
            pl.BlockSpec((_K6B_TILE, OUT), lambda ph, t: (t, 0)),
            pl.BlockSpec((_K6B_TILE, 1), lambda ph, t: (t, 0)),
            pl.BlockSpec((OUT, MLP_HID), lambda ph, t: (0, 0)),
            pl.BlockSpec((MLP_HID, MLP_HID), lambda ph, t: (0, 0)),
            pl.BlockSpec((MLP_HID, MLP_HID), lambda ph, t: (0, 0)),
            pl.BlockSpec((MLP_HID, OUT), lambda ph, t: (0, 0)),
            pl.BlockSpec((3, MLP_HID), lambda ph, t: (0, 0)),
            pl.BlockSpec((3, MLP_HID), lambda ph, t: (0, 0)),
            pl.BlockSpec((OUT, 16), lambda ph, t: (0, 0)),
            pl.BlockSpec((16,), lambda ph, t: (0,)),
            pl.BlockSpec((16, ACT), lambda ph, t: (0, 0)),
            pl.BlockSpec((ACT,), lambda ph, t: (0,)),
        ],
        out_specs=pl.BlockSpec((G, ACT), lambda ph, t: (0, 0)),
        out_shape=jax.ShapeDtypeStruct((G, ACT), jnp.float32),
        scratch_shapes=[
            pltpu.VMEM((N, MLP_HID), jnp.float32),
            pltpu.VMEM((3, MLP_HID), jnp.float32),
            pltpu.VMEM((3, MLP_HID), jnp.float32),
            pltpu.VMEM((G, OUT), jnp.float32),
            pltpu.VMEM((1, G), jnp.float32),
        ],
    )(out, batch2d, mlp1_W, mlp2_W, mlp3_W, mlp4_W, bn_gamma, bn_beta,
      ain_W, ain_b, aout_W, aout_b)


def kernel(x, edge_index, edge_attr, batch, node_W, node_b, edge_W, edge_b,
           src_W, dst_W, lin_edge_W, mlp1_W, mlp2_W, mlp3_W, mlp4_W, bn_gamma,
           bn_beta, ain_W, ain_b, aout_W, aout_b):
    hsq, hd, maxh = _node_encode(x, node_W, node_b, src_W, dst_W)

    w2 = edge_W @ lin_edge_W
    b2 = edge_b @ lin_edge_W
    epq, maxe = _edge_encode(edge_attr, w2, b2)

    m_shift = jnp.maximum(maxh[0, 0] + maxe[0, 0], 0.0) + EPS
    mvec = jnp.full((16,), m_shift, jnp.float32)
    zeros = jnp.zeros((NPAD, 32), jnp.float32)

    src = edge_index[0]
    dst = edge_index[1]
    o = _softmax_aggregate(hsq, epq, src, dst, mvec, zeros)

    out = _combine(o, hd)
    batch2d = batch.reshape(N, 1).astype(jnp.int32)
    return _mlp_head(out, batch2d, mlp1_W, mlp2_W, mlp3_W, mlp4_W, bn_gamma,
                     bn_beta, ain_W, ain_b, aout_W, aout_b)


# speedup vs baseline: 6.5835x; 1.0578x over previous
import functools

import jax
import jax.numpy as jnp
from jax import lax
from jax.experimental import pallas as pl
from jax.experimental.pallas import tpu as pltpu
from jax.experimental.pallas import tpu_sc as plsc

N = 50000
E = 800000
G = 64
D_IN = 128
D_EDGE = 16
HID = 32
OUT = 64
MLP_HID = 128
ACT = 16
EPS = 1e-7

NQ = 4
NPAD = 50176
ROWS_PER_TILE = NPAD // 16
CH = 128
NCHUNKS = E // CH
NJ = 392
EP8 = E // 8


_K1_TILE = 2000


def _k1_body(x_ref, nw_ref, nb_ref, sw_ref, dw_ref, hsq_ref, hd_ref, mx_ref,
             acc_ref):
    t = pl.program_id(0)
    h = jnp.dot(x_ref[...], nw_ref[...], preferred_element_type=jnp.float32)
    h = h + nb_ref[...]
    hs = jnp.dot(h, sw_ref[...], preferred_element_type=jnp.float32)
    hd = jnp.dot(h, dw_ref[...], preferred_element_type=jnp.float32)
    hd_ref[...] = hd
    for q in range(NQ):
        hsq_ref[q] = hs[:, 16 * q:16 * (q + 1)]
    m = jnp.max(hs)

    @pl.when(t == 0)
    def _():
        acc_ref[0] = m

    @pl.when(t > 0)
    def _():
        acc_ref[0] = jnp.maximum(acc_ref[0], m)

    @pl.when(t == pl.num_programs(0) - 1)
    def _():
        mx_ref[0, 0] = acc_ref[0]


def _node_encode(x, node_W, node_b, src_W, dst_W):
    grid = (N // _K1_TILE,)
    hsq, hd, mx = pl.pallas_call(
        _k1_body,
        grid=grid,
        in_specs=[
            pl.BlockSpec((_K1_TILE, D_IN), lambda t: (t, 0)),
            pl.BlockSpec((D_IN, HID), lambda t: (0, 0)),
            pl.BlockSpec((HID,), lambda t: (0,)),
            pl.BlockSpec((HID, OUT), lambda t: (0, 0)),
            pl.BlockSpec((HID, OUT), lambda t: (0, 0)),
        ],
        out_specs=[
            pl.BlockSpec((NQ, _K1_TILE, 16), lambda t: (0, t, 0)),
            pl.BlockSpec((_K1_TILE, OUT), lambda t: (t, 0)),
            pl.BlockSpec(memory_space=pltpu.SMEM),
        ],
        out_shape=[
            jax.ShapeDtypeStruct((NQ, N, 16), jnp.float32),
            jax.ShapeDtypeStruct((N, OUT), jnp.float32),
            jax.ShapeDtypeStruct((1, 1), jnp.float32),
        ],
        scratch_shapes=[pltpu.SMEM((1,), jnp.float32)],
    )(x, node_W, node_b, src_W, dst_W)
    return hsq, hd, mx


_K4_TILE8 = 1000


def _k4_body(ea_ref, w_ref, b_ref, epq_ref, mx_ref, acc_ref):
    t = pl.program_id(0)
    x = ea_ref[...]
    m = None
    for q in range(NQ):
        ep = jnp.dot(x, w_ref[q], preferred_element_type=jnp.float32)
        ep = ep + b_ref[q]
        epq_ref[q] = ep
        mq = jnp.max(ep)
        m = mq if m is None else jnp.maximum(m, mq)

    @pl.when(t == 0)
    def _():
        acc_ref[0] = m

    @pl.when(t > 0)
    def _():
        acc_ref[0] = jnp.maximum(acc_ref[0], m)

    @pl.when(t == pl.num_programs(0) - 1)
    def _():
        mx_ref[0, 0] = acc_ref[0]


def _edge_encode(edge_attr, w2, b2):
    wblk = jnp.zeros((NQ, 128, 128), jnp.float32)
    for q in range(NQ):
        for u in range(8):
            wblk = wblk.at[q, 16 * u:16 * (u + 1),
                           16 * u:16 * (u + 1)].set(w2[:, 16 * q:16 * (q + 1)])
    bpack = jnp.tile(b2.reshape(NQ, 16), (1, 8))

    ea8 = edge_attr.reshape(EP8, 128)
    grid = (EP8 // _K4_TILE8,)
    epq, mx = pl.pallas_call(
        _k4_body,
        grid=grid,
        in_specs=[
            pl.BlockSpec((_K4_TILE8, 128), lambda t: (t, 0)),
            pl.BlockSpec((NQ, 128, 128), lambda t: (0, 0, 0)),
            pl.BlockSpec((NQ, 128), lambda t: (0, 0)),
        ],
        out_specs=[
            pl.BlockSpec((NQ, _K4_TILE8, 128), lambda t: (0, t, 0)),
            pl.BlockSpec(memory_space=pltpu.SMEM),
        ],
        out_shape=[
            jax.ShapeDtypeStruct((NQ, EP8, 128), jnp.float32),
            jax.ShapeDtypeStruct((1, 1), jnp.float32),
        ],
        scratch_shapes=[pltpu.SMEM((1,), jnp.float32)],
    )(ea8, wblk, bpack)
    return epq, mx


def _k5_body(hsf, epq, src_h, dst_h, mvec_h, zeros_h, out_h,
             accum, mbuf,
             sidx0, sidx1, didx0, didx1, ebuf0, ebuf1,
             gbuf0, gbuf1, vbuf0, vbuf1, cidx0, cidx1,
             sem_i0, sem_i1, sem_g, sem_s0, sem_s1):
    cid = lax.axis_index("c")
    sid = lax.axis_index("s")

    sidx = (sidx0, sidx1)
    didx = (didx0, didx1)
    ebuf = (ebuf0, ebuf1)
    gbuf = (gbuf0, gbuf1)
    vbuf = (vbuf0, vbuf1)
    cidx = (cidx0, cidx1)
    sem_i = (sem_i0, sem_i1)
    sem_s = (sem_s0, sem_s1)

    pltpu.sync_copy(mvec_h, mbuf)
    mv = mbuf[...]
    lane = lax.iota(jnp.int32, 16)

    def chunk_off(j):
        cidr = sid + j * 16
        return jnp.minimum(cidr, NCHUNKS - 1) * CH, cidr < NCHUNKS

    for r in range(2):
        q = 2 * r + cid
        qoff = q * N
        pltpu.sync_copy(zeros_h.at[pl.ds(sid * ROWS_PER_TILE, ROWS_PER_TILE)],
                        accum.at[pl.ds(sid * ROWS_PER_TILE, ROWS_PER_TILE)])
        plsc.subcore_barrier()

        def issue_in(j, b):
            off, _ = chunk_off(j)
            pltpu.async_copy(src_h.at[pl.ds(off, CH)], sidx[b], sem_i[b])
            pltpu.async_copy(dst_h.at[pl.ds(off, CH)], didx[b], sem_i[b])
            pltpu.async_copy(epq.at[q, pl.ds(off // 8, CH // 8)], ebuf[b],
                             sem_i[b])

        def wait_in(j, b):
            off, _ = chunk_off(j)
            pltpu.make_async_copy(src_h.at[pl.ds(off, CH)], sidx[b],
                                  sem_i[b]).wait()
            pltpu.make_async_copy(dst_h.at[pl.ds(off, CH)], didx[b],
                                  sem_i[b]).wait()
            pltpu.make_async_copy(epq.at[q, pl.ds(off // 8, CH // 8)],
                                  ebuf[b], sem_i[b]).wait()

        def arm_gather(b):
            for i in range(CH // 16):
                sidx[b][pl.ds(i * 16, 16)] = \
                    sidx[b][pl.ds(i * 16, 16)] + qoff
            pltpu.async_copy(hsf.at[sidx[b]], gbuf[b], sem_g)

        issue_in(0, 0)
        issue_in(1, 1)
        wait_in(0, 0)
        arm_gather(0)

        def step(j, b):
            nb = 1 - b
            off, valid = chunk_off(j)
            pltpu.make_async_copy(hsf.at[sidx[b]], gbuf[b], sem_g).wait()

            @pl.when(j + 1 < NJ)
            def _():
                wait_in(j + 1, nb)
                arm_gather(nb)

            @pl.when(j >= 2)
            def _():
                pltpu.make_async_copy(vbuf[b], accum.at[cidx[b]],
                                      sem_s[b]).wait()

            shift = jnp.where(valid, 0, N + 16)
            vs = [didx[b][pl.ds(i * 16, 16)] + shift
                  for i in range(CH // 16)]
            lidxs = [jnp.where(v < N, v, N + lane) for v in vs]
            for i in range(CH // 16):
                cidx[b][pl.ds(i * 16, 16)] = lidxs[i]

            def vstep(i, _):
                gs = [gbuf[b][i * 8 + u] for u in range(8)]
                eas = [ebuf[b][i, pl.ds(u * 16, 16)] for u in range(8)]
                msgs = [jnp.maximum(g + a, 0.0) + EPS
                        for g, a in zip(gs, eas)]
                exs = [jnp.exp(m - mv) for m in msgs]
                ems = [x * m for x, m in zip(exs, msgs)]
                for u in range(8):
                    vbuf[b][i * 8 + u, 0:16] = exs[u]
                for u in range(8):
                    vbuf[b][i * 8 + u, 16:32] = ems[u]
                return 0

            lax.fori_loop(0, CH // 8, vstep, 0)

            pltpu.async_copy(vbuf[b], accum.at[cidx[b]], sem_s[b], add=True)

            @pl.when(j + 2 < NJ)
            def _():
                issue_in(j + 2, b)

        def loop_body(g2, _):
            step(2 * g2, 0)
            step(2 * g2 + 1, 1)
            return 0

        lax.fori_loop(0, NJ // 2, loop_body, 0)

        pltpu.make_async_copy(vbuf[0], accum.at[cidx[0]], sem_s[0]).wait()
        pltpu.make_async_copy(vbuf[1], accum.at[cidx[1]], sem_s[1]).wait()
        plsc.subcore_barrier()
        pltpu.sync_copy(accum.at[pl.ds(sid * ROWS_PER_TILE, ROWS_PER_TILE)],
                        out_h.at[q, pl.ds(sid * ROWS_PER_TILE,
                                          ROWS_PER_TILE)])
        plsc.subcore_barrier()


def _softmax_aggregate(hsq, epq, src, dst, mvec, zeros):
    mesh = plsc.VectorSubcoreMesh(core_axis_name="c", subcore_axis_name="s")
    kfun = pl.kernel(
        _k5_body,
        out_type=jax.ShapeDtypeStruct((NQ, NPAD, 32), jnp.float32),
        mesh=mesh,
        compiler_params=pltpu.CompilerParams(use_tc_tiling_on_sc=False),
        scratch_types=[
            pltpu.VMEM_SHARED((NPAD, 32), jnp.float32),
            pltpu.VMEM((16,), jnp.float32),
            pltpu.VMEM((CH,), jnp.int32),
            pltpu.VMEM((CH,), jnp.int32),
            pltpu.VMEM((CH,), jnp.int32),
            pltpu.VMEM((CH,), jnp.int32),
            pltpu.VMEM((CH // 8, 128), jnp.float32),
            pltpu.VMEM((CH // 8, 128), jnp.float32),
            pltpu.VMEM((CH, 16), jnp.float32),
            pltpu.VMEM((CH, 16), jnp.float32),
            pltpu.VMEM((CH, 32), jnp.float32),
            pltpu.VMEM((CH, 32), jnp.float32),
            pltpu.VMEM((CH,), jnp.int32),
            pltpu.VMEM((CH,), jnp.int32),
            pltpu.SemaphoreType.DMA,
            pltpu.SemaphoreType.DMA,
            pltpu.SemaphoreType.DMA,
            pltpu.SemaphoreType.DMA,
            pltpu.SemaphoreType.DMA,
        ],
    )
    hsf = hsq.reshape(NQ * N, 16)
    return kfun(hsf, epq, src, dst, mvec, zeros)


_K6B_TILE = 2000
_K6B_NT = N // _K6B_TILE


def _k6b_body(o_ref, hd_ref, batch_ref, w1_ref, w2_ref, w3_ref, w4_ref,
              gam_ref, bet_ref, ainw_ref, ainb_ref, aoutw_ref, aoutb_ref,
              a_ref, hh_ref, s_ref, ss_ref, pool_ref, cnt_ref):
    ph = pl.program_id(0)
    t = pl.program_id(1)
    nt = pl.num_programs(1)
    row = pl.ds(t * _K6B_TILE, _K6B_TILE)
    inv_n = 1.0 / N

    def bn(x, l):
        mu = s_ref[l] * inv_n
        var = ss_ref[l] * inv_n - mu * mu
        xn = (x - mu) * lax.rsqrt(var + 1e-5)
        return jnp.maximum(xn * gam_ref[l] + bet_ref[l], 0.0)

    def stats(l, y):
        @pl.when(t == 0)
        def _():
            s_ref[l] = jnp.sum(y, axis=0)
            ss_ref[l] = jnp.sum(y * y, axis=0)

        @pl.when(t > 0)
        def _():
            s_ref[l] = s_ref[l] + jnp.sum(y, axis=0)
            ss_ref[l] = ss_ref[l] + jnp.sum(y * y, axis=0)

    @pl.when(ph == 0)
    def _():
        cols = []
        for q in range(NQ):
            den = o_ref[q, :, 0:16]
            num = o_ref[q, :, 16:32]
            cols.append(jnp.where(den > 0.0, num / den, 0.0))
        x = jnp.concatenate(cols, axis=1) + hd_ref[...]
        y = jnp.dot(x, w1_ref[...], preferred_element_type=jnp.float32)
        hh_ref[row, :] = y
        stats(0, y)

    @pl.when(ph == 1)
    def _():
        y = jnp.dot(bn(hh_ref[row, :], 0), w2_ref[...],
                    preferred_element_type=jnp.float32)
        hh_ref[row, :] = y
        stats(1, y)

    @pl.when(ph == 2)
    def _():
        y = jnp.dot(bn(hh_ref[row, :], 1), w3_ref[...],
                    preferred_element_type=jnp.float32)
        hh_ref[row, :] = y
        stats(2, y)

    @pl.when(ph == 3)
    def _():
        p = jnp.dot(bn(hh_ref[row, :], 2), w4_ref[...],
                    preferred_element_type=jnp.float32)
        gids = lax.broadcasted_iota(jnp.int32, (_K6B_TILE, G), 1)
        onehot = (batch_ref[...] == gids).astype(jnp.float32)
        psum = jnp.dot(onehot.T, p, preferred_element_type=jnp.float32)
        csum = jnp.sum(onehot, axis=0)

        @pl.when(t == 0)
        def _():
            pool_ref[...] = psum
            cnt_ref[0] = csum

        @pl.when(t > 0)
        def _():
            pool_ref[...] = pool_ref[...] + psum
            cnt_ref[0] = cnt_ref[0] + csum

        @pl.when(t == nt - 1)
        def _():
            cnt = jnp.maximum(cnt_ref[0], 1.0)
            mol = pool_ref[...] / cnt[:, None]
            z = jnp.maximum(
                jnp.dot(mol, ainw_ref[...],
                        preferred_element_type=jnp.float32) + ainb_ref[...],
                0.0)
            a_ref[...] = jnp.tanh(
                jnp.dot(z, aoutw_ref[...],
                        preferred_element_type=jnp.float32) + aoutb_ref[...])


def _mlp_head(o, hd, batch2d, mlp1_W, mlp2_W, mlp3_W, mlp4_W, bn_gamma,
              bn_beta, ain_W, ain_b, aout_W, aout_b):
    return pl.pallas_call(
        _k6b_body,
        grid=(4, _K6B_NT),
        in_specs=[
            pl.BlockSpec((NQ, _K6B_TILE, 32),
                         lambda ph, t: (0, jnp.where(ph == 0, t, 0), 0)),
            pl.BlockSpec((_K6B_TILE, OUT),
                         lambda ph, t: (jnp.where(ph == 0, t, 0), 0)),
            pl.BlockSpec((_K6B_TILE, 1), lambda ph, t: (t, 0)),
            pl.BlockSpec((OUT, MLP_HID), lambda ph, t: (0, 0)),
            pl.BlockSpec((MLP_HID, MLP_HID), lambda ph, t: (0, 0)),
            pl.BlockSpec((MLP_HID, MLP_HID), lambda ph, t: (0, 0)),
            pl.BlockSpec((MLP_HID, OUT), lambda ph, t: (0, 0)),
            pl.BlockSpec((3, MLP_HID), lambda ph, t: (0, 0)),
            pl.BlockSpec((3, MLP_HID), lambda ph, t: (0, 0)),
            pl.BlockSpec((OUT, 16), lambda ph, t: (0, 0)),
            pl.BlockSpec((16,), lambda ph, t: (0,)),
            pl.BlockSpec((16, ACT), lambda ph, t: (0, 0)),
            pl.BlockSpec((ACT,), lambda ph, t: (0,)),
        ],
        out_specs=pl.BlockSpec((G, ACT), lambda ph, t: (0, 0)),
        out_shape=jax.ShapeDtypeStruct((G, ACT), jnp.float32),
        scratch_shapes=[
            pltpu.VMEM((N, MLP_HID), jnp.float32),
            pltpu.VMEM((3, MLP_HID), jnp.float32),
            pltpu.VMEM((3, MLP_HID), jnp.float32),
            pltpu.VMEM((G, OUT), jnp.float32),
            pltpu.VMEM((1, G), jnp.float32),
        ],
    )(o, hd, batch2d, mlp1_W, mlp2_W, mlp3_W, mlp4_W, bn_gamma, bn_beta,
      ain_W, ain_b, aout_W, aout_b)


def kernel(x, edge_index, edge_attr, batch, node_W, node_b, edge_W, edge_b,
           src_W, dst_W, lin_edge_W, mlp1_W, mlp2_W, mlp3_W, mlp4_W, bn_gamma,
           bn_beta, ain_W, ain_b, aout_W, aout_b):
    hsq, hd, maxh = _node_encode(x, node_W, node_b, src_W, dst_W)

    w2 = edge_W @ lin_edge_W
    b2 = edge_b @ lin_edge_W
    epq, maxe = _edge_encode(edge_attr, w2, b2)

    m_shift = jnp.maximum(maxh[0, 0] + maxe[0, 0], 0.0) + EPS
    mvec = jnp.full((16,), m_shift, jnp.float32)
    zeros = jnp.zeros((NPAD, 32), jnp.float32)

    src = edge_index[0]
    dst = edge_index[1]
    o = _softmax_aggregate(hsq, epq, src, dst, mvec, zeros)

    batch2d = batch.reshape(N, 1).astype(jnp.int32)
    return _mlp_head(o, hd, batch2d, mlp1_W, mlp2_W, mlp3_W, mlp4_W,
                     bn_gamma, bn_beta, ain_W, ain_b, aout_W, aout_b)


# speedup vs baseline: 6.7245x; 1.0214x over previous
import functools

import jax
import jax.numpy as jnp
from jax import lax
from jax.experimental import pallas as pl
from jax.experimental.pallas import tpu as pltpu
from jax.experimental.pallas import tpu_sc as plsc

N = 50000
E = 800000
G = 64
D_IN = 128
D_EDGE = 16
HID = 32
OUT = 64
MLP_HID = 128
ACT = 16
EPS = 1e-7

NQ = 4
NPAD = 50176
ROWS_PER_TILE = NPAD // 16
CH = 128
NCHUNKS = E // CH
NJ = 392
EP8 = E // 8


_K1_TILE = 2000


def _k1_body(x_ref, nw_ref, nb_ref, sw_ref, dw_ref, hsq_ref, hd_ref, mx_ref,
             acc_ref):
    t = pl.program_id(0)
    h = jnp.dot(x_ref[...], nw_ref[...], preferred_element_type=jnp.float32)
    h = h + nb_ref[...]
    hs = jnp.dot(h, sw_ref[...], preferred_element_type=jnp.float32)
    hd = jnp.dot(h, dw_ref[...], preferred_element_type=jnp.float32)
    hd_ref[...] = hd
    for q in range(NQ):
        hsq_ref[q] = hs[:, 16 * q:16 * (q + 1)]
    m = jnp.max(hs)

    @pl.when(t == 0)
    def _():
        acc_ref[0] = m

    @pl.when(t > 0)
    def _():
        acc_ref[0] = jnp.maximum(acc_ref[0], m)

    @pl.when(t == pl.num_programs(0) - 1)
    def _():
        mx_ref[0, 0] = acc_ref[0]


def _node_encode(x, node_W, node_b, src_W, dst_W):
    grid = (N // _K1_TILE,)
    hsq, hd, mx = pl.pallas_call(
        _k1_body,
        grid=grid,
        in_specs=[
            pl.BlockSpec((_K1_TILE, D_IN), lambda t: (t, 0)),
            pl.BlockSpec((D_IN, HID), lambda t: (0, 0)),
            pl.BlockSpec((HID,), lambda t: (0,)),
            pl.BlockSpec((HID, OUT), lambda t: (0, 0)),
            pl.BlockSpec((HID, OUT), lambda t: (0, 0)),
        ],
        out_specs=[
            pl.BlockSpec((NQ, _K1_TILE, 16), lambda t: (0, t, 0)),
            pl.BlockSpec((_K1_TILE, OUT), lambda t: (t, 0)),
            pl.BlockSpec(memory_space=pltpu.SMEM),
        ],
        out_shape=[
            jax.ShapeDtypeStruct((NQ, N, 16), jnp.float32),
            jax.ShapeDtypeStruct((N, OUT), jnp.float32),
            jax.ShapeDtypeStruct((1, 1), jnp.float32),
        ],
        scratch_shapes=[pltpu.SMEM((1,), jnp.float32)],
    )(x, node_W, node_b, src_W, dst_W)
    return hsq, hd, mx


_K4_TILE8 = 1000


def _k4_body(ea_ref, w_ref, b_ref, epq_ref, mx_ref, acc_ref):
    t = pl.program_id(0)
    x = ea_ref[...]
    m = None
    for q in range(2):
        ep = jnp.dot(x, w_ref[q], preferred_element_type=jnp.float32)
        ep = ep + b_ref[q]
        epq_ref[q] = ep
        mq = jnp.max(ep)
        m = mq if m is None else jnp.maximum(m, mq)

    @pl.when(t == 0)
    def _():
        acc_ref[0] = m

    @pl.when(t > 0)
    def _():
        acc_ref[0] = jnp.maximum(acc_ref[0], m)

    @pl.when(t == pl.num_programs(0) - 1)
    def _():
        mx_ref[0, 0] = acc_ref[0]


def _edge_encode(ea8, w2, b2, qbase):
    wblk = jnp.zeros((2, 128, 128), jnp.float32)
    for i in range(2):
        q = qbase + i
        for u in range(8):
            wblk = wblk.at[i, 16 * u:16 * (u + 1),
                           16 * u:16 * (u + 1)].set(w2[:, 16 * q:16 * (q + 1)])
    bpack = jnp.tile(b2.reshape(NQ, 16)[qbase:qbase + 2], (1, 8))

    grid = (EP8 // _K4_TILE8,)
    epq, mx = pl.pallas_call(
        _k4_body,
        grid=grid,
        in_specs=[
            pl.BlockSpec((_K4_TILE8, 128), lambda t: (t, 0)),
            pl.BlockSpec((2, 128, 128), lambda t: (0, 0, 0)),
            pl.BlockSpec((2, 128), lambda t: (0, 0)),
        ],
        out_specs=[
            pl.BlockSpec((2, _K4_TILE8, 128), lambda t: (0, t, 0)),
            pl.BlockSpec(memory_space=pltpu.SMEM),
        ],
        out_shape=[
            jax.ShapeDtypeStruct((2, EP8, 128), jnp.float32),
            jax.ShapeDtypeStruct((1, 1), jnp.float32),
        ],
        scratch_shapes=[pltpu.SMEM((1,), jnp.float32)],
    )(ea8, wblk, bpack)
    return epq, mx


def _k5_body(qbase, hsf, epq, src_h, dst_h, mvec_h, zeros_h, out_h,
             accum, mbuf,
             sidx0, sidx1, didx0, didx1, ebuf0, ebuf1,
             gbuf0, gbuf1, vbuf0, vbuf1, cidx0, cidx1,
             sem_i0, sem_i1, sem_g, sem_s0, sem_s1):
    cid = lax.axis_index("c")
    sid = lax.axis_index("s")

    sidx = (sidx0, sidx1)
    didx = (didx0, didx1)
    ebuf = (ebuf0, ebuf1)
    gbuf = (gbuf0, gbuf1)
    vbuf = (vbuf0, vbuf1)
    cidx = (cidx0, cidx1)
    sem_i = (sem_i0, sem_i1)
    sem_s = (sem_s0, sem_s1)

    pltpu.sync_copy(mvec_h, mbuf)
    mv = mbuf[...]
    lane = lax.iota(jnp.int32, 16)

    def chunk_off(j):
        cidr = sid + j * 16
        return jnp.minimum(cidr, NCHUNKS - 1) * CH, cidr < NCHUNKS

    for r in range(1):
        qoff = (qbase + cid) * N
        pltpu.sync_copy(zeros_h.at[pl.ds(sid * ROWS_PER_TILE, ROWS_PER_TILE)],
                        accum.at[pl.ds(sid * ROWS_PER_TILE, ROWS_PER_TILE)])
        plsc.subcore_barrier()

        def issue_in(j, b):
            off, _ = chunk_off(j)
            pltpu.async_copy(src_h.at[pl.ds(off, CH)], sidx[b], sem_i[b])
            pltpu.async_copy(dst_h.at[pl.ds(off, CH)], didx[b], sem_i[b])
            pltpu.async_copy(epq.at[cid, pl.ds(off // 8, CH // 8)],
                             ebuf[b], sem_i[b])

        def wait_in(j, b):
            off, _ = chunk_off(j)
            pltpu.make_async_copy(src_h.at[pl.ds(off, CH)], sidx[b],
                                  sem_i[b]).wait()
            pltpu.make_async_copy(dst_h.at[pl.ds(off, CH)], didx[b],
                                  sem_i[b]).wait()
            pltpu.make_async_copy(epq.at[cid, pl.ds(off // 8, CH // 8)],
                                  ebuf[b], sem_i[b]).wait()

        def arm_gather(b):
            for i in range(CH // 16):
                sidx[b][pl.ds(i * 16, 16)] = \
                    sidx[b][pl.ds(i * 16, 16)] + qoff
            pltpu.async_copy(hsf.at[sidx[b]], gbuf[b], sem_g)

        issue_in(0, 0)
        issue_in(1, 1)
        wait_in(0, 0)
        arm_gather(0)

        def step(j, b):
            nb = 1 - b
            off, valid = chunk_off(j)
            pltpu.make_async_copy(hsf.at[sidx[b]], gbuf[b], sem_g).wait()

            @pl.when(j + 1 < NJ)
            def _():
                wait_in(j + 1, nb)
                arm_gather(nb)

            @pl.when(j >= 2)
            def _():
                pltpu.make_async_copy(vbuf[b], accum.at[cidx[b]],
                                      sem_s[b]).wait()

            shift = jnp.where(valid, 0, N + 16)
            vs = [didx[b][pl.ds(i * 16, 16)] + shift
                  for i in range(CH // 16)]
            lidxs = [jnp.where(v < N, v, N + lane) for v in vs]
            for i in range(CH // 16):
                cidx[b][pl.ds(i * 16, 16)] = lidxs[i]

            def vstep(i, _):
                gs = [gbuf[b][i * 8 + u] for u in range(8)]
                eas = [ebuf[b][i, pl.ds(u * 16, 16)] for u in range(8)]
                msgs = [jnp.maximum(g + a, 0.0) + EPS
                        for g, a in zip(gs, eas)]
                exs = [jnp.exp(m - mv) for m in msgs]
                ems = [x * m for x, m in zip(exs, msgs)]
                for u in range(8):
                    vbuf[b][i * 8 + u, 0:16] = exs[u]
                for u in range(8):
                    vbuf[b][i * 8 + u, 16:32] = ems[u]
                return 0

            lax.fori_loop(0, CH // 8, vstep, 0)

            pltpu.async_copy(vbuf[b], accum.at[cidx[b]], sem_s[b], add=True)

            @pl.when(j + 2 < NJ)
            def _():
                issue_in(j + 2, b)

        def loop_body(g2, _):
            step(2 * g2, 0)
            step(2 * g2 + 1, 1)
            return 0

        lax.fori_loop(0, NJ // 2, loop_body, 0)

        pltpu.make_async_copy(vbuf[0], accum.at[cidx[0]], sem_s[0]).wait()
        pltpu.make_async_copy(vbuf[1], accum.at[cidx[1]], sem_s[1]).wait()
        plsc.subcore_barrier()
        pltpu.sync_copy(accum.at[pl.ds(sid * ROWS_PER_TILE, ROWS_PER_TILE)],
                        out_h.at[cid, pl.ds(sid * ROWS_PER_TILE,
                                            ROWS_PER_TILE)])
        plsc.subcore_barrier()


def _softmax_aggregate(hsf, epq, src, dst, mvec, zeros, qbase):
    mesh = plsc.VectorSubcoreMesh(core_axis_name="c", subcore_axis_name="s")
    kfun = pl.kernel(
        functools.partial(_k5_body, qbase),
        out_type=jax.ShapeDtypeStruct((2, NPAD, 32), jnp.float32),
        mesh=mesh,
        compiler_params=pltpu.CompilerParams(use_tc_tiling_on_sc=False),
        scratch_types=[
            pltpu.VMEM_SHARED((NPAD, 32), jnp.float32),
            pltpu.VMEM((16,), jnp.float32),
            pltpu.VMEM((CH,), jnp.int32),
            pltpu.VMEM((CH,), jnp.int32),
            pltpu.VMEM((CH,), jnp.int32),
            pltpu.VMEM((CH,), jnp.int32),
            pltpu.VMEM((CH // 8, 128), jnp.float32),
            pltpu.VMEM((CH // 8, 128), jnp.float32),
            pltpu.VMEM((CH, 16), jnp.float32),
            pltpu.VMEM((CH, 16), jnp.float32),
            pltpu.VMEM((CH, 32), jnp.float32),
            pltpu.VMEM((CH, 32), jnp.float32),
            pltpu.VMEM((CH,), jnp.int32),
            pltpu.VMEM((CH,), jnp.int32),
            pltpu.SemaphoreType.DMA,
            pltpu.SemaphoreType.DMA,
            pltpu.SemaphoreType.DMA,
            pltpu.SemaphoreType.DMA,
            pltpu.SemaphoreType.DMA,
        ],
    )
    return kfun(hsf, epq, src, dst, mvec, zeros)


_K6B_TILE = 2000
_K6B_NT = N // _K6B_TILE


def _k6b_body(o01_ref, o23_ref, hd_ref, batch_ref, w1_ref, w2_ref, w3_ref,
              w4_ref,
              gam_ref, bet_ref, ainw_ref, ainb_ref, aoutw_ref, aoutb_ref,
              a_ref, hh_ref, s_ref, ss_ref, pool_ref, cnt_ref):
    ph = pl.program_id(0)
    t = pl.program_id(1)
    nt = pl.num_programs(1)
    row = pl.ds(t * _K6B_TILE, _K6B_TILE)
    inv_n = 1.0 / N

    def bn(x, l):
        mu = s_ref[l] * inv_n
        var = ss_ref[l] * inv_n - mu * mu
        xn = (x - mu) * lax.rsqrt(var + 1e-5)
        return jnp.maximum(xn * gam_ref[l] + bet_ref[l], 0.0)

    def stats(l, y):
        @pl.when(t == 0)
        def _():
            s_ref[l] = jnp.sum(y, axis=0)
            ss_ref[l] = jnp.sum(y * y, axis=0)

        @pl.when(t > 0)
        def _():
            s_ref[l] = s_ref[l] + jnp.sum(y, axis=0)
            ss_ref[l] = ss_ref[l] + jnp.sum(y * y, axis=0)

    @pl.when(ph == 0)
    def _():
        cols = []
        for q in range(NQ):
            oref = o01_ref if q < 2 else o23_ref
            den = oref[q % 2, :, 0:16]
            num = oref[q % 2, :, 16:32]
            cols.append(jnp.where(den > 0.0, num / den, 0.0))
        x = jnp.concatenate(cols, axis=1) + hd_ref[...]
        y = jnp.dot(x, w1_ref[...], preferred_element_type=jnp.float32)
        hh_ref[row, :] = y
        stats(0, y)

    @pl.when(ph == 1)
    def _():
        y = jnp.dot(bn(hh_ref[row, :], 0), w2_ref[...],
                    preferred_element_type=jnp.float32)
        hh_ref[row, :] = y
        stats(1, y)

    @pl.when(ph == 2)
    def _():
        y = jnp.dot(bn(hh_ref[row, :], 1), w3_ref[...],
                    preferred_element_type=jnp.float32)
        hh_ref[row, :] = y
        stats(2, y)

    @pl.when(ph == 3)
    def _():
        p = jnp.dot(bn(hh_ref[row, :], 2), w4_ref[...],
                    preferred_element_type=jnp.float32)
        gids = lax.broadcasted_iota(jnp.int32, (_K6B_TILE, G), 1)
        onehot = (batch_ref[...] == gids).astype(jnp.float32)
        psum = jnp.dot(onehot.T, p, preferred_element_type=jnp.float32)
        csum = jnp.sum(onehot, axis=0)

        @pl.when(t == 0)
        def _():
            pool_ref[...] = psum
            cnt_ref[0] = csum

        @pl.when(t > 0)
        def _():
            pool_ref[...] = pool_ref[...] + psum
            cnt_ref[0] = cnt_ref[0] + csum

        @pl.when(t == nt - 1)
        def _():
            cnt = jnp.maximum(cnt_ref[0], 1.0)
            mol = pool_ref[...] / cnt[:, None]
            z = jnp.maximum(
                jnp.dot(mol, ainw_ref[...],
                        preferred_element_type=jnp.float32) + ainb_ref[...],
                0.0)
            a_ref[...] = jnp.tanh(
                jnp.dot(z, aoutw_ref[...],
                        preferred_element_type=jnp.float32) + aoutb_ref[...])


def _mlp_head(o01, o23, hd, batch2d, mlp1_W, mlp2_W, mlp3_W, mlp4_W,
              bn_gamma, bn_beta, ain_W, ain_b, aout_W, aout_b):
    return pl.pallas_call(
        _k6b_body,
        grid=(4, _K6B_NT),
        in_specs=[
            pl.BlockSpec((2, _K6B_TILE, 32),
                         lambda ph, t: (0, jnp.where(ph == 0, t, 0), 0)),
            pl.BlockSpec((2, _K6B_TILE, 32),
                         lambda ph, t: (0, jnp.where(ph == 0, t, 0), 0)),
            pl.BlockSpec((_K6B_TILE, OUT),
                         lambda ph, t: (jnp.where(ph == 0, t, 0), 0)),
            pl.BlockSpec((_K6B_TILE, 1), lambda ph, t: (t, 0)),
            pl.BlockSpec((OUT, MLP_HID), lambda ph, t: (0, 0)),
            pl.BlockSpec((MLP_HID, MLP_HID), lambda ph, t: (0, 0)),
            pl.BlockSpec((MLP_HID, MLP_HID), lambda ph, t: (0, 0)),
            pl.BlockSpec((MLP_HID, OUT), lambda ph, t: (0, 0)),
            pl.BlockSpec((3, MLP_HID), lambda ph, t: (0, 0)),
            pl.BlockSpec((3, MLP_HID), lambda ph, t: (0, 0)),
            pl.BlockSpec((OUT, 16), lambda ph, t: (0, 0)),
            pl.BlockSpec((16,), lambda ph, t: (0,)),
            pl.BlockSpec((16, ACT), lambda ph, t: (0, 0)),
            pl.BlockSpec((ACT,), lambda ph, t: (0,)),
        ],
        out_specs=pl.BlockSpec((G, ACT), lambda ph, t: (0, 0)),
        out_shape=jax.ShapeDtypeStruct((G, ACT), jnp.float32),
        scratch_shapes=[
            pltpu.VMEM((N, MLP_HID), jnp.float32),
            pltpu.VMEM((3, MLP_HID), jnp.float32),
            pltpu.VMEM((3, MLP_HID), jnp.float32),
            pltpu.VMEM((G, OUT), jnp.float32),
            pltpu.VMEM((1, G), jnp.float32),
        ],
    )(o01, o23, hd, batch2d, mlp1_W, mlp2_W, mlp3_W, mlp4_W, bn_gamma,
      bn_beta, ain_W, ain_b, aout_W, aout_b)


def kernel(x, edge_index, edge_attr, batch, node_W, node_b, edge_W, edge_b,
           src_W, dst_W, lin_edge_W, mlp1_W, mlp2_W, mlp3_W, mlp4_W, bn_gamma,
           bn_beta, ain_W, ain_b, aout_W, aout_b):
    hsq, hd, maxh = _node_encode(x, node_W, node_b, src_W, dst_W)
    hsf = hsq.reshape(NQ * N, 16)

    w2 = edge_W @ lin_edge_W
    b2 = edge_b @ lin_edge_W
    ea8 = edge_attr.reshape(EP8, 128)
    zeros = jnp.zeros((NPAD, 32), jnp.float32)
    src = edge_index[0]
    dst = edge_index[1]

    ep01, maxe01 = _edge_encode(ea8, w2, b2, 0)
    m01 = jnp.maximum(maxh[0, 0] + maxe01[0, 0], 0.0) + EPS
    o01 = _softmax_aggregate(hsf, ep01, src, dst,
                             jnp.full((16,), m01, jnp.float32), zeros, 0)

    ep23, maxe23 = _edge_encode(ea8, w2, b2, 2)
    m23 = jnp.maximum(maxh[0, 0] + maxe23[0, 0], 0.0) + EPS
    o23 = _softmax_aggregate(hsf, ep23, src, dst,
                             jnp.full((16,), m23, jnp.float32), zeros, 2)

    batch2d = batch.reshape(N, 1).astype(jnp.int32)
    return _mlp_head(o01, o23, hd, batch2d, mlp1_W, mlp2_W, mlp3_W, mlp4_W,
                     bn_gamma, bn_beta, ain_W, ain_b, aout_W, aout_b)


# speedup vs baseline: 6.9649x; 1.0358x over previous
import functools

import jax
import jax.numpy as jnp
from jax import lax
from jax.experimental import pallas as pl
from jax.experimental.pallas import tpu as pltpu
from jax.experimental.pallas import tpu_sc as plsc

N = 50000
E = 800000
G = 64
D_IN = 128
D_EDGE = 16
HID = 32
OUT = 64
MLP_HID = 128
ACT = 16
EPS = 1e-7

NQ = 4
NPAD = 50176
ROWS_PER_TILE = NPAD // 16
CH = 128
NCHUNKS = E // CH
NJ = 392
EP8 = E // 8


_K1_TILE = 2000


def _k1_body(x_ref, nw_ref, nb_ref, sw_ref, dw_ref, hsq_ref, hd_ref, mx_ref,
             acc_ref):
    t = pl.program_id(0)
    h = jnp.dot(x_ref[...], nw_ref[...], preferred_element_type=jnp.float32)
    h = h + nb_ref[...]
    hs = jnp.dot(h, sw_ref[...], preferred_element_type=jnp.float32)
    hd = jnp.dot(h, dw_ref[...], preferred_element_type=jnp.float32)
    hd_ref[...] = hd
    for q in range(NQ):
        hsq_ref[q] = hs[:, 16 * q:16 * (q + 1)]
    m = jnp.max(hs)

    @pl.when(t == 0)
    def _():
        acc_ref[0] = m

    @pl.when(t > 0)
    def _():
        acc_ref[0] = jnp.maximum(acc_ref[0], m)

    @pl.when(t == pl.num_programs(0) - 1)
    def _():
        mx_ref[0, 0] = acc_ref[0]


def _node_encode(x, node_W, node_b, src_W, dst_W):
    grid = (N // _K1_TILE,)
    hsq, hd, mx = pl.pallas_call(
        _k1_body,
        grid=grid,
        in_specs=[
            pl.BlockSpec((_K1_TILE, D_IN), lambda t: (t, 0)),
            pl.BlockSpec((D_IN, HID), lambda t: (0, 0)),
            pl.BlockSpec((HID,), lambda t: (0,)),
            pl.BlockSpec((HID, OUT), lambda t: (0, 0)),
            pl.BlockSpec((HID, OUT), lambda t: (0, 0)),
        ],
        out_specs=[
            pl.BlockSpec((NQ, _K1_TILE, 16), lambda t: (0, t, 0)),
            pl.BlockSpec((_K1_TILE, OUT), lambda t: (t, 0)),
            pl.BlockSpec(memory_space=pltpu.SMEM),
        ],
        out_shape=[
            jax.ShapeDtypeStruct((NQ, N, 16), jnp.float32),
            jax.ShapeDtypeStruct((N, OUT), jnp.float32),
            jax.ShapeDtypeStruct((1, 1), jnp.float32),
        ],
        scratch_shapes=[pltpu.SMEM((1,), jnp.float32)],
    )(x, node_W, node_b, src_W, dst_W)
    return hsq, hd, mx


_K4_TILE8 = 1000


def _k4_body(ea_ref, w_ref, b_ref, epq_ref, mx_ref, acc_ref):
    t = pl.program_id(0)
    x = ea_ref[...]
    m = None
    for q in range(2):
        ep = jnp.dot(x, w_ref[q], preferred_element_type=jnp.float32)
        ep = ep + b_ref[q]
        epq_ref[q] = ep
        mq = jnp.max(ep)
        m = mq if m is None else jnp.maximum(m, mq)

    @pl.when(t == 0)
    def _():
        acc_ref[0] = m

    @pl.when(t > 0)
    def _():
        acc_ref[0] = jnp.maximum(acc_ref[0], m)

    @pl.when(t == pl.num_programs(0) - 1)
    def _():
        mx_ref[0, 0] = acc_ref[0]


def _edge_encode(ea8, w2, b2, qbase):
    wblk = jnp.zeros((2, 128, 128), jnp.float32)
    for i in range(2):
        q = qbase + i
        for u in range(8):
            wblk = wblk.at[i, 16 * u:16 * (u + 1),
                           16 * u:16 * (u + 1)].set(w2[:, 16 * q:16 * (q + 1)])
    bpack = jnp.tile(b2.reshape(NQ, 16)[qbase:qbase + 2], (1, 8))

    grid = (EP8 // _K4_TILE8,)
    epq, mx = pl.pallas_call(
        _k4_body,
        grid=grid,
        in_specs=[
            pl.BlockSpec((_K4_TILE8, 128), lambda t: (t, 0)),
            pl.BlockSpec((2, 128, 128), lambda t: (0, 0, 0)),
            pl.BlockSpec((2, 128), lambda t: (0, 0)),
        ],
        out_specs=[
            pl.BlockSpec((2, _K4_TILE8, 128), lambda t: (0, t, 0)),
            pl.BlockSpec(memory_space=pltpu.SMEM),
        ],
        out_shape=[
            jax.ShapeDtypeStruct((2, EP8, 128), jnp.float32),
            jax.ShapeDtypeStruct((1, 1), jnp.float32),
        ],
        scratch_shapes=[pltpu.SMEM((1,), jnp.float32)],
    )(ea8, wblk, bpack)
    return epq, mx


def _k5_body(qbase, hsf, epq, src_h, dst_h, mvec_h, zeros_h, out_h,
             accum, mbuf,
             sidx0, sidx1, didx0, didx1, ebuf0, ebuf1,
             gbuf0, gbuf1, vbuf0, vbuf1, cidx0, cidx1,
             sem_i0, sem_i1, sem_g, sem_s0, sem_s1):
    cid = lax.axis_index("c")
    sid = lax.axis_index("s")

    sidx = (sidx0, sidx1)
    didx = (didx0, didx1)
    ebuf = (ebuf0, ebuf1)
    gbuf = (gbuf0, gbuf1)
    vbuf = (vbuf0, vbuf1)
    cidx = (cidx0, cidx1)
    sem_i = (sem_i0, sem_i1)
    sem_s = (sem_s0, sem_s1)

    pltpu.sync_copy(mvec_h, mbuf)
    mv = mbuf[...]
    lane = lax.iota(jnp.int32, 16)

    def chunk_off(j):
        cidr = sid + j * 16
        return jnp.minimum(cidr, NCHUNKS - 1) * CH, cidr < NCHUNKS

    for r in range(1):
        qoff = (qbase + cid) * N
        pltpu.sync_copy(zeros_h.at[pl.ds(sid * ROWS_PER_TILE, ROWS_PER_TILE)],
                        accum.at[pl.ds(sid * ROWS_PER_TILE, ROWS_PER_TILE)])
        plsc.subcore_barrier()

        def issue_in(j, b):
            off, _ = chunk_off(j)
            pltpu.async_copy(src_h.at[pl.ds(off, CH)], sidx[b], sem_i[b])
            pltpu.async_copy(dst_h.at[pl.ds(off, CH)], didx[b], sem_i[b])
            pltpu.async_copy(epq.at[cid, pl.ds(off // 8, CH // 8)],
                             ebuf[b], sem_i[b])

        def wait_in(j, b):
            off, _ = chunk_off(j)
            pltpu.make_async_copy(src_h.at[pl.ds(off, CH)], sidx[b],
                                  sem_i[b]).wait()
            pltpu.make_async_copy(dst_h.at[pl.ds(off, CH)], didx[b],
                                  sem_i[b]).wait()
            pltpu.make_async_copy(epq.at[cid, pl.ds(off // 8, CH // 8)],
                                  ebuf[b], sem_i[b]).wait()

        def arm_gather(b):
            for i in range(CH // 16):
                sidx[b][pl.ds(i * 16, 16)] = \
                    sidx[b][pl.ds(i * 16, 16)] + qoff
            pltpu.async_copy(hsf.at[sidx[b]], gbuf[b], sem_g)

        issue_in(0, 0)
        issue_in(1, 1)
        wait_in(0, 0)
        arm_gather(0)

        def step(j, b):
            nb = 1 - b
            off, valid = chunk_off(j)
            pltpu.make_async_copy(hsf.at[sidx[b]], gbuf[b], sem_g).wait()

            @pl.when(j + 1 < NJ)
            def _():
                wait_in(j + 1, nb)
                arm_gather(nb)

            @pl.when(j >= 2)
            def _():
                pltpu.make_async_copy(vbuf[b], accum.at[cidx[b]],
                                      sem_s[b]).wait()

            shift = jnp.where(valid, 0, N + 16)

            def vstep(i, _):
                v = didx[b][pl.ds(i * 16, 16)] + shift
                cidx[b][pl.ds(i * 16, 16)] = \
                    jnp.where(v < N, v, N + lane)
                gs = [gbuf[b][i * 16 + u] for u in range(16)]
                eas = [ebuf[b][2 * i + u // 8, pl.ds((u % 8) * 16, 16)]
                       for u in range(16)]
                msgs = [jnp.maximum(g + a, 0.0) + EPS
                        for g, a in zip(gs, eas)]
                exs = [jnp.exp(m - mv) for m in msgs]
                ems = [x * m for x, m in zip(exs, msgs)]
                for u in range(16):
                    vbuf[b][i * 16 + u, 0:16] = exs[u]
                for u in range(16):
                    vbuf[b][i * 16 + u, 16:32] = ems[u]
                return 0

            lax.fori_loop(0, CH // 16, vstep, 0)

            pltpu.async_copy(vbuf[b], accum.at[cidx[b]], sem_s[b], add=True)

            @pl.when(j + 2 < NJ)
            def _():
                issue_in(j + 2, b)

        def loop_body(g2, _):
            step(2 * g2, 0)
            step(2 * g2 + 1, 1)
            return 0

        lax.fori_loop(0, NJ // 2, loop_body, 0)

        pltpu.make_async_copy(vbuf[0], accum.at[cidx[0]], sem_s[0]).wait()
        pltpu.make_async_copy(vbuf[1], accum.at[cidx[1]], sem_s[1]).wait()
        plsc.subcore_barrier()
        pltpu.sync_copy(accum.at[pl.ds(sid * ROWS_PER_TILE, ROWS_PER_TILE)],
                        out_h.at[cid, pl.ds(sid * ROWS_PER_TILE,
                                            ROWS_PER_TILE)])
        plsc.subcore_barrier()


def _softmax_aggregate(hsf, epq, src, dst, mvec, zeros, qbase):
    mesh = plsc.VectorSubcoreMesh(core_axis_name="c", subcore_axis_name="s")
    kfun = pl.kernel(
        functools.partial(_k5_body, qbase),
        out_type=jax.ShapeDtypeStruct((2, NPAD, 32), jnp.float32),
        mesh=mesh,
        compiler_params=pltpu.CompilerParams(use_tc_tiling_on_sc=False),
        scratch_types=[
            pltpu.VMEM_SHARED((NPAD, 32), jnp.float32),
            pltpu.VMEM((16,), jnp.float32),
            pltpu.VMEM((CH,), jnp.int32),
            pltpu.VMEM((CH,), jnp.int32),
            pltpu.VMEM((CH,), jnp.int32),
            pltpu.VMEM((CH,), jnp.int32),
            pltpu.VMEM((CH // 8, 128), jnp.float32),
            pltpu.VMEM((CH // 8, 128), jnp.float32),
            pltpu.VMEM((CH, 16), jnp.float32),
            pltpu.VMEM((CH, 16), jnp.float32),
            pltpu.VMEM((CH, 32), jnp.float32),
            pltpu.VMEM((CH, 32), jnp.float32),
            pltpu.VMEM((CH,), jnp.int32),
            pltpu.VMEM((CH,), jnp.int32),
            pltpu.SemaphoreType.DMA,
            pltpu.SemaphoreType.DMA,
            pltpu.SemaphoreType.DMA,
            pltpu.SemaphoreType.DMA,
            pltpu.SemaphoreType.DMA,
        ],
    )
    return kfun(hsf, epq, src, dst, mvec, zeros)


_K6B_TILE = 2000
_K6B_NT = N // _K6B_TILE


def _k6b_body(o01_ref, o23_ref, hd_ref, batch_ref, w1_ref, w2_ref, w3_ref,
              w4_ref,
              gam_ref, bet_ref, ainw_ref, ainb_ref, aoutw_ref, aoutb_ref,
              a_ref, hh_ref, s_ref, ss_ref, pool_ref, cnt_ref):
    ph = pl.program_id(0)
    t = pl.program_id(1)
    nt = pl.num_programs(1)
    row = pl.ds(t * _K6B_TILE, _K6B_TILE)
    inv_n = 1.0 / N

    def bn(x, l):
        mu = s_ref[l] * inv_n
        var = ss_ref[l] * inv_n - mu * mu
        xn = (x - mu) * lax.rsqrt(var + 1e-5)
        return jnp.maximum(xn * gam_ref[l] + bet_ref[l], 0.0)

    def stats(l, y):
        @pl.when(t == 0)
        def _():
            s_ref[l] = jnp.sum(y, axis=0)
            ss_ref[l] = jnp.sum(y * y, axis=0)

        @pl.when(t > 0)
        def _():
            s_ref[l] = s_ref[l] + jnp.sum(y, axis=0)
            ss_ref[l] = ss_ref[l] + jnp.sum(y * y, axis=0)

    @pl.when(ph == 0)
    def _():
        cols = []
        for q in range(NQ):
            oref = o01_ref if q < 2 else o23_ref
            den = oref[q % 2, :, 0:16]
            num = oref[q % 2, :, 16:32]
            cols.append(jnp.where(den > 0.0, num / den, 0.0))
        x = jnp.concatenate(cols, axis=1) + hd_ref[...]
        y = jnp.dot(x, w1_ref[...], preferred_element_type=jnp.float32)
        hh_ref[row, :] = y
        stats(0, y)

    @pl.when(ph == 1)
    def _():
        y = jnp.dot(bn(hh_ref[row, :], 0), w2_ref[...],
                    preferred_element_type=jnp.float32)
        hh_ref[row, :] = y
        stats(1, y)

    @pl.when(ph == 2)
    def _():
        y = jnp.dot(bn(hh_ref[row, :], 1), w3_ref[...],
                    preferred_element_type=jnp.float32)
        hh_ref[row, :] = y
        stats(2, y)

    @pl.when(ph == 3)
    def _():
        p = jnp.dot(bn(hh_ref[row, :], 2), w4_ref[...],
                    preferred_element_type=jnp.float32)
        gids = lax.broadcasted_iota(jnp.int32, (_K6B_TILE, G), 1)
        onehot = (batch_ref[...] == gids).astype(jnp.float32)
        psum = jnp.dot(onehot.T, p, preferred_element_type=jnp.float32)
        csum = jnp.sum(onehot, axis=0)

        @pl.when(t == 0)
        def _():
            pool_ref[...] = psum
            cnt_ref[0] = csum

        @pl.when(t > 0)
        def _():
            pool_ref[...] = pool_ref[...] + psum
            cnt_ref[0] = cnt_ref[0] + csum

        @pl.when(t == nt - 1)
        def _():
            cnt = jnp.maximum(cnt_ref[0], 1.0)
            mol = pool_ref[...] / cnt[:, None]
            z = jnp.maximum(
                jnp.dot(mol, ainw_ref[...],
                        preferred_element_type=jnp.float32) + ainb_ref[...],
                0.0)
            a_ref[...] = jnp.tanh(
                jnp.dot(z, aoutw_ref[...],
                        preferred_element_type=jnp.float32) + aoutb_ref[...])


def _mlp_head(o01, o23, hd, batch2d, mlp1_W, mlp2_W, mlp3_W, mlp4_W,
              bn_gamma, bn_beta, ain_W, ain_b, aout_W, aout_b):
    return pl.pallas_call(
        _k6b_body,
        grid=(4, _K6B_NT),
        in_specs=[
            pl.BlockSpec((2, _K6B_TILE, 32),
                         lambda ph, t: (0, jnp.where(ph == 0, t, 0), 0)),
            pl.BlockSpec((2, _K6B_TILE, 32),
                         lambda ph, t: (0, jnp.where(ph == 0, t, 0), 0)),
            pl.BlockSpec((_K6B_TILE, OUT),
                         lambda ph, t: (jnp.where(ph == 0, t, 0), 0)),
            pl.BlockSpec((_K6B_TILE, 1), lambda ph, t: (t, 0)),
            pl.BlockSpec((OUT, MLP_HID), lambda ph, t: (0, 0)),
            pl.BlockSpec((MLP_HID, MLP_HID), lambda ph, t: (0, 0)),
            pl.BlockSpec((MLP_HID, MLP_HID), lambda ph, t: (0, 0)),
            pl.BlockSpec((MLP_HID, OUT), lambda ph, t: (0, 0)),
            pl.BlockSpec((3, MLP_HID), lambda ph, t: (0, 0)),
            pl.BlockSpec((3, MLP_HID), lambda ph, t: (0, 0)),
            pl.BlockSpec((OUT, 16), lambda ph, t: (0, 0)),
            pl.BlockSpec((16,), lambda ph, t: (0,)),
            pl.BlockSpec((16, ACT), lambda ph, t: (0, 0)),
            pl.BlockSpec((ACT,), lambda ph, t: (0,)),
        ],
        out_specs=pl.BlockSpec((G, ACT), lambda ph, t: (0, 0)),
        out_shape=jax.ShapeDtypeStruct((G, ACT), jnp.float32),
        scratch_shapes=[
            pltpu.VMEM((N, MLP_HID), jnp.float32),
            pltpu.VMEM((3, MLP_HID), jnp.float32),
            pltpu.VMEM((3, MLP_HID), jnp.float32),
            pltpu.VMEM((G, OUT), jnp.float32),
            pltpu.VMEM((1, G), jnp.float32),
        ],
    )(o01, o23, hd, batch2d, mlp1_W, mlp2_W, mlp3_W, mlp4_W, bn_gamma,
      bn_beta, ain_W, ain_b, aout_W, aout_b)


def kernel(x, edge_index, edge_attr, batch, node_W, node_b, edge_W, edge_b,
           src_W, dst_W, lin_edge_W, mlp1_W, mlp2_W, mlp3_W, mlp4_W, bn_gamma,
           bn_beta, ain_W, ain_b, aout_W, aout_b):
    hsq, hd, maxh = _node_encode(x, node_W, node_b, src_W, dst_W)
    hsf = hsq.reshape(NQ * N, 16)

    w2 = edge_W @ lin_edge_W
    b2 = edge_b @ lin_edge_W
    ea8 = edge_attr.reshape(EP8, 128)
    zeros = jnp.zeros((NPAD, 32), jnp.float32)
    src = edge_index[0]
    dst = edge_index[1]

    ep01, maxe01 = _edge_encode(ea8, w2, b2, 0)
    m01 = jnp.maximum(maxh[0, 0] + maxe01[0, 0], 0.0) + EPS
    o01 = _softmax_aggregate(hsf, ep01, src, dst,
                             jnp.full((16,), m01, jnp.float32), zeros, 0)

    ep23, maxe23 = _edge_encode(ea8, w2, b2, 2)
    m23 = jnp.maximum(maxh[0, 0] + maxe23[0, 0], 0.0) + EPS
    o23 = _softmax_aggregate(hsf, ep23, src, dst,
                             jnp.full((16,), m23, jnp.float32), zeros, 2)

    batch2d = batch.reshape(N, 1).astype(jnp.int32)
    return _mlp_head(o01, o23, hd, batch2d, mlp1_W, mlp2_W, mlp3_W, mlp4_W,
                     bn_gamma, bn_beta, ain_W, ain_b, aout_W, aout_b)
